# baseline (speedup 1.0000x reference)
import jax
import jax.numpy as jnp
from jax import lax
from jax.experimental import pallas as pl

P = 32
V_BLOCK = 512


def _pointnet_block(pts_ref, cnt_ref, w1_ref, w2_ref, w3_ref, b3_ref,
                    out_ref):
    V = out_ref.shape[0]
    cnt = cnt_ref[...]
    pts_t = pts_ref[...]

    h1 = lax.dot_general(pts_t, w1_ref[...],
                         dimension_numbers=(((0,), (0,)), ((), ())),
                         preferred_element_type=jnp.float32)
    h1 = jnp.maximum(h1, 0.0)

    h13 = h1.reshape(P, V, 33)
    empty33 = jnp.broadcast_to(cnt <= 0, (V, 33))
    slab0 = lax.broadcasted_iota(jnp.int32, (P, V, 33), 0) == 0
    zero_h1 = jnp.maximum(w1_ref[3:4, :].astype(jnp.float32), 0.0)
    h13 = jnp.where(slab0 & empty33[None], zero_h1[None], h13)
    h1 = h13.reshape(P * V, 33)

    h2 = jnp.dot(h1, w2_ref[...], preferred_element_type=jnp.float32)
    h2 = jnp.maximum(h2, 0.0)

    C = 128
    cnt_b = jnp.broadcast_to(jnp.maximum(cnt, 1), (V, C))
    mask = lax.broadcasted_iota(jnp.int32, (P, V, C), 0) < cnt_b[None]
    for k in range(256 // C):
        h3 = jnp.dot(h2, w3_ref[:, k * C:(k + 1) * C],
                     preferred_element_type=jnp.float32)
        h3 = h3.reshape(P, V, C)
        masked = jnp.where(mask, h3, jnp.float32(-1e30))
        out_ref[:, k * C:(k + 1) * C] = (jnp.max(masked, axis=0)
                                         + b3_ref[:, k * C:(k + 1) * C])


def kernel(Frustum_Voxel, Frustum_Voxel_num, W1, b1, W2, b2, W3, b3):
    B, H, Wd, Pp, _ = Frustum_Voxel.shape
    N = B * H * Wd
    nb = N // V_BLOCK

    fv16 = Frustum_Voxel.astype(jnp.bfloat16)
    t = fv16.reshape(nb, V_BLOCK, Pp, 3).transpose(0, 2, 1, 3)
    pts_t = t.reshape(nb * Pp * V_BLOCK, 3).T
    pts_t = jnp.concatenate(
        [pts_t, jnp.ones((1, pts_t.shape[1]), jnp.bfloat16)], axis=0)
    cnt = Frustum_Voxel_num.reshape(N, 1)

    w1a = jnp.zeros((4, 33), jnp.float32)
    w1a = w1a.at[:3, :32].set(W1).at[3, :32].set(b1).at[3, 32].set(1.0)
    w1a = w1a.astype(jnp.bfloat16)
    w2a = jnp.concatenate([W2, b2.reshape(1, 256)], axis=0)

    feat = pl.pallas_call(
        _pointnet_block,
        grid=(nb,),
        in_specs=[
            pl.BlockSpec((4, Pp * V_BLOCK), lambda i: (0, i)),
            pl.BlockSpec((V_BLOCK, 1), lambda i: (i, 0)),
            pl.BlockSpec((4, 33), lambda i: (0, 0)),
            pl.BlockSpec((33, 256), lambda i: (0, 0)),
            pl.BlockSpec((256, 256), lambda i: (0, 0)),
            pl.BlockSpec((1, 256), lambda i: (0, 0)),
        ],
        out_specs=pl.BlockSpec((V_BLOCK, 256), lambda i: (i, 0)),
        out_shape=jax.ShapeDtypeStruct((N, 256), jnp.float32),
    )(pts_t, cnt, w1a, w2a, W3, b3.reshape(1, 256))

    return feat.reshape(B, H, Wd, 256)


# speedup vs baseline: 1.2844x; 1.2844x over previous
import jax
import jax.numpy as jnp
from jax import lax
from jax.experimental import pallas as pl

P = 32
V_BLOCK = 512


def _pointnet_block(pts_ref, cnt_ref, w1_ref, w2_ref, w3_ref, b3_ref,
                    out_ref):
    V = out_ref.shape[0]
    cnt = cnt_ref[...]
    pts_t = pts_ref[...]

    h1 = lax.dot_general(pts_t, w1_ref[...],
                         dimension_numbers=(((0,), (0,)), ((), ())),
                         preferred_element_type=jnp.float32)
    h1 = jnp.maximum(h1, 0.0)

    h13 = h1.reshape(P, V, 33)
    empty33 = jnp.broadcast_to(cnt <= 0, (V, 33))
    slab0 = lax.broadcasted_iota(jnp.int32, (P, V, 33), 0) == 0
    zero_h1 = jnp.maximum(w1_ref[3:4, :].astype(jnp.float32), 0.0)
    h13 = jnp.where(slab0 & empty33[None], zero_h1[None], h13)
    h1 = h13.reshape(P * V, 33)

    h2 = jnp.dot(h1, w2_ref[...], preferred_element_type=jnp.float32)
    h2 = jnp.maximum(h2, 0.0)

    C = P // 4
    cnt_b = jnp.broadcast_to(jnp.maximum(cnt, 1), (V, 256))
    parts = []
    for k in range(P // C):
        h3 = jnp.dot(h2[k * C * V:(k + 1) * C * V],
                     w3_ref[...], preferred_element_type=jnp.float32)
        h3 = h3.reshape(C, V, 256)
        pid = lax.broadcasted_iota(jnp.int32, (C, V, 256), 0) + (k * C)
        parts.append(jnp.max(
            jnp.where(pid < cnt_b[None], h3, jnp.float32(-1e30)), axis=0))
    m = jnp.maximum(jnp.maximum(parts[0], parts[1]),
                    jnp.maximum(parts[2], parts[3]))
    out_ref[...] = m + b3_ref[...]


def kernel(Frustum_Voxel, Frustum_Voxel_num, W1, b1, W2, b2, W3, b3):
    B, H, Wd, Pp, _ = Frustum_Voxel.shape
    N = B * H * Wd
    nb = N // V_BLOCK

    fv16 = Frustum_Voxel.astype(jnp.bfloat16)
    t = fv16.reshape(nb, V_BLOCK, Pp, 3).transpose(0, 2, 1, 3)
    pts_t = t.reshape(nb * Pp * V_BLOCK, 3).T
    pts_t = jnp.concatenate(
        [pts_t, jnp.ones((1, pts_t.shape[1]), jnp.bfloat16)], axis=0)
    cnt = Frustum_Voxel_num.reshape(N, 1)

    w1a = jnp.zeros((4, 33), jnp.float32)
    w1a = w1a.at[:3, :32].set(W1).at[3, :32].set(b1).at[3, 32].set(1.0)
    w1a = w1a.astype(jnp.bfloat16)
    w2a = jnp.concatenate([W2, b2.reshape(1, 256)], axis=0)

    feat = pl.pallas_call(
        _pointnet_block,
        grid=(nb,),
        in_specs=[
            pl.BlockSpec((4, Pp * V_BLOCK), lambda i: (0, i)),
            pl.BlockSpec((V_BLOCK, 1), lambda i: (i, 0)),
            pl.BlockSpec((4, 33), lambda i: (0, 0)),
            pl.BlockSpec((33, 256), lambda i: (0, 0)),
            pl.BlockSpec((256, 256), lambda i: (0, 0)),
            pl.BlockSpec((1, 256), lambda i: (0, 0)),
        ],
        out_specs=pl.BlockSpec((V_BLOCK, 256), lambda i: (i, 0)),
        out_shape=jax.ShapeDtypeStruct((N, 256), jnp.float32),
    )(pts_t, cnt, w1a, w2a, W3, b3.reshape(1, 256))

    return feat.reshape(B, H, Wd, 256)


# speedup vs baseline: 1.3255x; 1.0320x over previous
import jax
import jax.numpy as jnp
from jax import lax
from jax.experimental import pallas as pl

P = 32
V_BLOCK = 512


def _pointnet_block(pts_ref, cnt_ref, w1_ref, w2_ref, w3_ref, b3_ref,
                    out_ref):
    V = out_ref.shape[0]
    cnt = cnt_ref[...]
    pts_t = pts_ref[...]

    h1 = lax.dot_general(pts_t, w1_ref[...],
                         dimension_numbers=(((0,), (0,)), ((), ())),
                         preferred_element_type=jnp.float32)
    h1 = jnp.maximum(h1, 0.0)

    h13 = h1.reshape(P, V, 33)
    empty33 = jnp.broadcast_to(cnt <= 0, (V, 33))
    slab0 = lax.broadcasted_iota(jnp.int32, (P, V, 33), 0) == 0
    zero_h1 = jnp.maximum(w1_ref[3:4, :].astype(jnp.float32), 0.0)
    h13 = jnp.where(slab0 & empty33[None], zero_h1[None], h13)
    h1 = h13.reshape(P * V, 33)

    h2 = jnp.dot(h1, w2_ref[...], preferred_element_type=jnp.float32)
    h2 = jnp.maximum(h2, 0.0)
    h3 = jnp.dot(h2, w3_ref[...], preferred_element_type=jnp.float32)
    h3 = h3.reshape(P, V, 256)

    cnt_b = jnp.broadcast_to(jnp.maximum(cnt, 1), (V, 256))
    mask = lax.broadcasted_iota(jnp.int32, (P, V, 256), 0) < cnt_b[None]
    masked = jnp.where(mask, h3, jnp.float32(-1e30))
    out_ref[...] = jnp.max(masked, axis=0) + b3_ref[...]


def kernel(Frustum_Voxel, Frustum_Voxel_num, W1, b1, W2, b2, W3, b3):
    B, H, Wd, Pp, _ = Frustum_Voxel.shape
    N = B * H * Wd
    nb = N // V_BLOCK

    fv16 = Frustum_Voxel.astype(jnp.bfloat16)
    t = fv16.reshape(nb, V_BLOCK, Pp, 3).transpose(0, 2, 1, 3)
    pts_t = t.reshape(nb * Pp * V_BLOCK, 3).T
    pts_t = jnp.concatenate(
        [pts_t, jnp.ones((1, pts_t.shape[1]), jnp.bfloat16)], axis=0)
    cnt = Frustum_Voxel_num.reshape(N, 1)

    w1a = jnp.zeros((4, 33), jnp.float32)
    w1a = w1a.at[:3, :32].set(W1).at[3, :32].set(b1).at[3, 32].set(1.0)
    w1a = w1a.astype(jnp.bfloat16)
    w2a = jnp.concatenate([W2, b2.reshape(1, 256)], axis=0)

    feat = pl.pallas_call(
        _pointnet_block,
        grid=(nb,),
        in_specs=[
            pl.BlockSpec((4, Pp * V_BLOCK), lambda i: (0, i)),
            pl.BlockSpec((V_BLOCK, 1), lambda i: (i, 0)),
            pl.BlockSpec((4, 33), lambda i: (0, 0)),
            pl.BlockSpec((33, 256), lambda i: (0, 0)),
            pl.BlockSpec((256, 256), lambda i: (0, 0)),
            pl.BlockSpec((1, 256), lambda i: (0, 0)),
        ],
        out_specs=pl.BlockSpec((V_BLOCK, 256), lambda i: (i, 0)),
        out_shape=jax.ShapeDtypeStruct((N, 256), jnp.float32),
    )(pts_t, cnt, w1a, w2a, W3, b3.reshape(1, 256))

    return feat.reshape(B, H, Wd, 256)
